# baseline (speedup 1.0000x reference)
import functools

import jax
import jax.numpy as jnp
from jax import lax
from jax.experimental import pallas as pl
from jax.experimental.pallas import tpu as pltpu
from jax.experimental.pallas import tpu_sc as plsc

B, N1, N2 = 4, 1024, 4096
C1, C2 = 256, 256
FEA_IN, FEA_OUT = 512, 256
NQ = B * N2

TILE_Q = 512
TILE_M = 1024

NW = 32
QPW = NQ // NW
CQ = 32


def _knn_body(p2_ref, p1t_ref, idx_ref, w_ref):
    b = pl.program_id(0)
    d2 = None
    for d in range(3):
        q = p2_ref[0, :, d:d + 1]
        r = p1t_ref[0, d:d + 1, :]
        diff = q - r
        sq = diff * diff
        d2 = sq if d2 is None else d2 + sq
    iota = lax.broadcasted_iota(jnp.int32, (TILE_Q, N1), 1).astype(jnp.float32)
    inf = jnp.float32(jnp.inf)
    vals, idxs = [], []
    d = d2
    for _ in range(3):
        m = jnp.min(d, axis=1, keepdims=True)
        i = jnp.min(jnp.where(d == m, iota, jnp.float32(N1)),
                    axis=1, keepdims=True)
        vals.append(m)
        idxs.append(i.astype(jnp.int32))
        d = jnp.where(iota == i, inf, d)
    r0 = 1.0 / (jnp.sqrt(vals[0]) + 1e-8)
    r1 = 1.0 / (jnp.sqrt(vals[1]) + 1e-8)
    r2 = 1.0 / (jnp.sqrt(vals[2]) + 1e-8)
    norm = r0 + r1 + r2
    w_ref[0] = jnp.concatenate(
        [jnp.broadcast_to(r0 / norm, (TILE_Q, 16)),
         jnp.broadcast_to(r1 / norm, (TILE_Q, 16)),
         jnp.broadcast_to(r2 / norm, (TILE_Q, 16))], axis=1)
    base = b * N1
    idx_ref[0] = jnp.concatenate(
        [idxs[0] + base, idxs[1] + base, idxs[2] + base], axis=1)


def _knn(p2, p1t):
    grid = (B, N2 // TILE_Q)
    return pl.pallas_call(
        _knn_body,
        grid=grid,
        in_specs=[
            pl.BlockSpec((1, TILE_Q, 3), lambda b, qt: (b, qt, 0)),
            pl.BlockSpec((1, 3, N1), lambda b, qt: (b, 0, 0)),
        ],
        out_specs=[
            pl.BlockSpec((1, TILE_Q, 3), lambda b, qt: (b, qt, 0)),
            pl.BlockSpec((1, TILE_Q, 48), lambda b, qt: (b, qt, 0)),
        ],
        out_shape=[
            jax.ShapeDtypeStruct((B, N2, 3), jnp.int32),
            jax.ShapeDtypeStruct((B, N2, 48), jnp.float32),
        ],
    )(p2, p1t)


def _gather_body(table_hbm, idx_hbm, wts_hbm, out_hbm,
                 idx_v, wts_v, rows_v, out_v, sem):
    wid = lax.axis_index("s") * 2 + lax.axis_index("c")
    qbase = wid * QPW

    def chunk(ch, _):
        q0 = qbase + ch * CQ
        pltpu.sync_copy(idx_hbm.at[pl.ds(q0 * 3, 3 * CQ)], idx_v)
        pltpu.sync_copy(wts_hbm.at[pl.ds(q0, CQ)], wts_v)
        pltpu.async_copy(table_hbm.at[idx_v], rows_v, sem).wait()

        for q in range(CQ):
            w0 = wts_v[q, pl.ds(0, 16)]
            w1 = wts_v[q, pl.ds(16, 16)]
            w2 = wts_v[q, pl.ds(32, 16)]
            for c in range(C1 // 16):
                sl = pl.ds(c * 16, 16)
                out_v[q, sl] = (rows_v[3 * q, sl] * w0
                                + rows_v[3 * q + 1, sl] * w1
                                + rows_v[3 * q + 2, sl] * w2)
        pltpu.sync_copy(out_v, out_hbm.at[pl.ds(q0, CQ)])
        return 0

    lax.fori_loop(0, QPW // CQ, chunk, 0)


@functools.lru_cache(maxsize=None)
def _make_gather_interp():
    return pl.kernel(
        _gather_body,
        out_type=jax.ShapeDtypeStruct((NQ, C1), jnp.float32),
        mesh=plsc.VectorSubcoreMesh(core_axis_name="c", subcore_axis_name="s"),
        scratch_types=[
            pltpu.VMEM((3 * CQ,), jnp.int32),
            pltpu.VMEM((CQ, 48), jnp.float32),
            pltpu.VMEM((3 * CQ, C1), jnp.float32),
            pltpu.VMEM((CQ, C1), jnp.float32),
            pltpu.SemaphoreType.DMA,
        ],
    )


def _mlp_body(interp_ref, x2_ref, w1a_ref, w1b_ref, t1_ref, w2_ref, t2_ref,
              out_ref):
    z = jnp.dot(interp_ref[...], w1a_ref[...],
                preferred_element_type=jnp.float32)
    z = z + jnp.dot(x2_ref[...], w1b_ref[...],
                    preferred_element_type=jnp.float32)
    h = jnp.maximum(z + t1_ref[...], 0.0)
    out_ref[...] = jnp.dot(h, w2_ref[...],
                           preferred_element_type=jnp.float32) + t2_ref[...]


def _mlp(interp, x2f, w1a, w1b, t1, w2t, t2):
    grid = (NQ // TILE_M,)
    return pl.pallas_call(
        _mlp_body,
        grid=grid,
        in_specs=[
            pl.BlockSpec((TILE_M, C1), lambda i: (i, 0)),
            pl.BlockSpec((TILE_M, C2), lambda i: (i, 0)),
            pl.BlockSpec((C1, FEA_OUT), lambda i: (0, 0)),
            pl.BlockSpec((C2, FEA_OUT), lambda i: (0, 0)),
            pl.BlockSpec((1, FEA_OUT), lambda i: (0, 0)),
            pl.BlockSpec((FEA_OUT, FEA_OUT), lambda i: (0, 0)),
            pl.BlockSpec((1, FEA_OUT), lambda i: (0, 0)),
        ],
        out_specs=pl.BlockSpec((TILE_M, FEA_OUT), lambda i: (i, 0)),
        out_shape=jax.ShapeDtypeStruct((NQ, FEA_OUT), jnp.float32),
    )(interp, x2f, w1a, w1b, t1, w2t, t2)


def kernel(x1, p1, x2, p2, W1, b1, g1, be1, m1, v1, W2, b2, g2, be2, m2, v2):
    p1t = jnp.swapaxes(p1, 1, 2)
    idx, wts = _knn(p2, p1t)

    table = x1.reshape(B * N1, C1)
    interp = _make_gather_interp()(table, idx.reshape(-1), wts.reshape(NQ, 48))

    s1 = g1 / jnp.sqrt(v1 + 1e-5)
    t1 = (b1 - m1) * s1 + be1
    s2 = g2 / jnp.sqrt(v2 + 1e-5)
    t2 = (b2 - m2) * s2 + be2
    w1s = W1 * s1[:, None]
    w1a = jnp.transpose(w1s[:, :C1])
    w1b = jnp.transpose(w1s[:, C1:])
    w2t = jnp.transpose(W2 * s2[:, None])

    h = _mlp(interp, x2.reshape(NQ, C2), w1a, w1b,
             t1.reshape(1, FEA_OUT), w2t, t2.reshape(1, FEA_OUT))
    return h.reshape(B, N2, FEA_OUT), p2


# speedup vs baseline: 1.1790x; 1.1790x over previous
import functools

import jax
import jax.numpy as jnp
from jax import lax
from jax.experimental import pallas as pl
from jax.experimental.pallas import tpu as pltpu
from jax.experimental.pallas import tpu_sc as plsc

B, N1, N2 = 4, 1024, 4096
C1, C2 = 256, 256
FEA_IN, FEA_OUT = 512, 256
NQ = B * N2

TILE_Q = 512
TILE_M = 1024

NW = 32
QPW = NQ // NW
CQ = 32


def _knn_body(p2_ref, p1t_ref, idx_ref, w_ref):
    b = pl.program_id(0)
    d2 = None
    for d in range(3):
        q = p2_ref[0, :, d:d + 1]
        r = p1t_ref[0, d:d + 1, :]
        diff = q - r
        sq = diff * diff
        d2 = sq if d2 is None else d2 + sq
    iota = lax.broadcasted_iota(jnp.int32, (TILE_Q, N1), 1).astype(jnp.float32)
    inf = jnp.float32(jnp.inf)
    vals, idxs = [], []
    d = d2
    for _ in range(3):
        m = jnp.min(d, axis=1, keepdims=True)
        i = jnp.min(jnp.where(d == m, iota, jnp.float32(N1)),
                    axis=1, keepdims=True)
        vals.append(m)
        idxs.append(i.astype(jnp.int32))
        d = jnp.where(iota == i, inf, d)
    r0 = 1.0 / (jnp.sqrt(vals[0]) + 1e-8)
    r1 = 1.0 / (jnp.sqrt(vals[1]) + 1e-8)
    r2 = 1.0 / (jnp.sqrt(vals[2]) + 1e-8)
    norm = r0 + r1 + r2
    w_ref[0] = jnp.concatenate(
        [jnp.broadcast_to(r0 / norm, (TILE_Q, 16)),
         jnp.broadcast_to(r1 / norm, (TILE_Q, 16)),
         jnp.broadcast_to(r2 / norm, (TILE_Q, 16))], axis=1)
    base = b * N1
    idx_ref[0] = jnp.concatenate(
        [idxs[0] + base, idxs[1] + base, idxs[2] + base], axis=1)


def _knn(p2, p1t):
    grid = (B, N2 // TILE_Q)
    return pl.pallas_call(
        _knn_body,
        grid=grid,
        in_specs=[
            pl.BlockSpec((1, TILE_Q, 3), lambda b, qt: (b, qt, 0)),
            pl.BlockSpec((1, 3, N1), lambda b, qt: (b, 0, 0)),
        ],
        out_specs=[
            pl.BlockSpec((1, TILE_Q, 3), lambda b, qt: (b, qt, 0)),
            pl.BlockSpec((1, TILE_Q, 48), lambda b, qt: (b, qt, 0)),
        ],
        out_shape=[
            jax.ShapeDtypeStruct((B, N2, 3), jnp.int32),
            jax.ShapeDtypeStruct((B, N2, 48), jnp.float32),
        ],
    )(p2, p1t)


def _gather_body(table_hbm, idx_hbm, wts_hbm, out_hbm,
                 idx_v0, idx_v1, wts_v0, wts_v1, rows_v0, rows_v1, out_v,
                 sem0, sem1):
    wid = lax.axis_index("s") * 2 + lax.axis_index("c")
    qbase = wid * QPW
    NCH = QPW // CQ
    idx_v = [idx_v0, idx_v1]
    wts_v = [wts_v0, wts_v1]
    rows_v = [rows_v0, rows_v1]
    sems = [sem0, sem1]

    def start(ch):
        q0 = qbase + ch * CQ
        s = ch % 2
        pltpu.sync_copy(idx_hbm.at[pl.ds(q0 * 3, 3 * CQ)], idx_v[s])
        pltpu.sync_copy(wts_hbm.at[pl.ds(q0, CQ)], wts_v[s])
        return pltpu.async_copy(table_hbm.at[idx_v[s]], rows_v[s], sems[s])

    cp = start(0)
    for ch in range(NCH):
        nxt_cp = start(ch + 1) if ch + 1 < NCH else None
        cp.wait()
        s = ch % 2
        wv, rv = wts_v[s], rows_v[s]

        def qbody(q, _):
            w0 = wv[q, pl.ds(0, 16)]
            w1 = wv[q, pl.ds(16, 16)]
            w2 = wv[q, pl.ds(32, 16)]
            for c in range(C1 // 16):
                sl = pl.ds(c * 16, 16)
                out_v[q, sl] = (rv[3 * q, sl] * w0
                                + rv[3 * q + 1, sl] * w1
                                + rv[3 * q + 2, sl] * w2)
            return 0

        lax.fori_loop(0, CQ, qbody, 0)
        pltpu.sync_copy(out_v, out_hbm.at[pl.ds(qbase + ch * CQ, CQ)])
        cp = nxt_cp


@functools.lru_cache(maxsize=None)
def _make_gather_interp():
    return pl.kernel(
        _gather_body,
        out_type=jax.ShapeDtypeStruct((NQ, C1), jnp.float32),
        mesh=plsc.VectorSubcoreMesh(core_axis_name="c", subcore_axis_name="s"),
        scratch_types=[
            pltpu.VMEM((3 * CQ,), jnp.int32),
            pltpu.VMEM((3 * CQ,), jnp.int32),
            pltpu.VMEM((CQ, 48), jnp.float32),
            pltpu.VMEM((CQ, 48), jnp.float32),
            pltpu.VMEM((3 * CQ, C1), jnp.float32),
            pltpu.VMEM((3 * CQ, C1), jnp.float32),
            pltpu.VMEM((CQ, C1), jnp.float32),
            pltpu.SemaphoreType.DMA,
            pltpu.SemaphoreType.DMA,
        ],
    )


def _mlp_body(interp_ref, x2_ref, w1a_ref, w1b_ref, t1_ref, w2_ref, t2_ref,
              out_ref):
    z = jnp.dot(interp_ref[...], w1a_ref[...],
                preferred_element_type=jnp.float32)
    z = z + jnp.dot(x2_ref[...], w1b_ref[...],
                    preferred_element_type=jnp.float32)
    h = jnp.maximum(z + t1_ref[...], 0.0)
    out_ref[...] = jnp.dot(h, w2_ref[...],
                           preferred_element_type=jnp.float32) + t2_ref[...]


def _mlp(interp, x2f, w1a, w1b, t1, w2t, t2):
    grid = (NQ // TILE_M,)
    return pl.pallas_call(
        _mlp_body,
        grid=grid,
        in_specs=[
            pl.BlockSpec((TILE_M, C1), lambda i: (i, 0)),
            pl.BlockSpec((TILE_M, C2), lambda i: (i, 0)),
            pl.BlockSpec((C1, FEA_OUT), lambda i: (0, 0)),
            pl.BlockSpec((C2, FEA_OUT), lambda i: (0, 0)),
            pl.BlockSpec((1, FEA_OUT), lambda i: (0, 0)),
            pl.BlockSpec((FEA_OUT, FEA_OUT), lambda i: (0, 0)),
            pl.BlockSpec((1, FEA_OUT), lambda i: (0, 0)),
        ],
        out_specs=pl.BlockSpec((TILE_M, FEA_OUT), lambda i: (i, 0)),
        out_shape=jax.ShapeDtypeStruct((NQ, FEA_OUT), jnp.float32),
    )(interp, x2f, w1a, w1b, t1, w2t, t2)


def kernel(x1, p1, x2, p2, W1, b1, g1, be1, m1, v1, W2, b2, g2, be2, m2, v2):
    p1t = jnp.swapaxes(p1, 1, 2)
    idx, wts = _knn(p2, p1t)

    table = x1.reshape(B * N1, C1)
    interp = _make_gather_interp()(table, idx.reshape(-1), wts.reshape(NQ, 48))

    s1 = g1 / jnp.sqrt(v1 + 1e-5)
    t1 = (b1 - m1) * s1 + be1
    s2 = g2 / jnp.sqrt(v2 + 1e-5)
    t2 = (b2 - m2) * s2 + be2
    w1s = W1 * s1[:, None]
    w1a = jnp.transpose(w1s[:, :C1])
    w1b = jnp.transpose(w1s[:, C1:])
    w2t = jnp.transpose(W2 * s2[:, None])

    h = _mlp(interp, x2.reshape(NQ, C2), w1a, w1b,
             t1.reshape(1, FEA_OUT), w2t, t2.reshape(1, FEA_OUT))
    return h.reshape(B, N2, FEA_OUT), p2


# speedup vs baseline: 1.4856x; 1.2601x over previous
import functools

import jax
import jax.numpy as jnp
from jax import lax
from jax.experimental import pallas as pl
from jax.experimental.pallas import tpu as pltpu
from jax.experimental.pallas import tpu_sc as plsc

B, N1, N2 = 4, 1024, 4096
C1, C2 = 256, 256
FEA_IN, FEA_OUT = 512, 256

TILE_Q = 512
TILE_M = 1024

NW = 32
QPW = N2 // NW
CQ = 32
NCH = QPW // CQ


def _make_knn_body(b):
    def _knn_body(p2_ref, p1t_ref, idx_ref, w_ref):
        d2 = None
        for d in range(3):
            q = p2_ref[0, :, d:d + 1]
            r = p1t_ref[0, d:d + 1, :]
            diff = q - r
            sq = diff * diff
            d2 = sq if d2 is None else d2 + sq
        iota = lax.broadcasted_iota(jnp.int32, (TILE_Q, N1), 1).astype(
            jnp.float32)
        inf = jnp.float32(jnp.inf)
        vals, idxs = [], []
        d = d2
        for _ in range(3):
            m = jnp.min(d, axis=1, keepdims=True)
            i = jnp.min(jnp.where(d == m, iota, jnp.float32(N1)),
                        axis=1, keepdims=True)
            vals.append(m)
            idxs.append(i.astype(jnp.int32))
            d = jnp.where(iota == i, inf, d)
        r0 = 1.0 / (jnp.sqrt(vals[0]) + 1e-8)
        r1 = 1.0 / (jnp.sqrt(vals[1]) + 1e-8)
        r2 = 1.0 / (jnp.sqrt(vals[2]) + 1e-8)
        norm = r0 + r1 + r2
        w_ref[...] = jnp.concatenate(
            [jnp.broadcast_to(r0 / norm, (TILE_Q, 16)),
             jnp.broadcast_to(r1 / norm, (TILE_Q, 16)),
             jnp.broadcast_to(r2 / norm, (TILE_Q, 16))], axis=1)
        base = b * N1
        idx_ref[...] = jnp.concatenate(
            [idxs[0] + base, idxs[1] + base, idxs[2] + base], axis=1)
    return _knn_body


def _knn(p2, p1t, b):
    return pl.pallas_call(
        _make_knn_body(b),
        grid=(N2 // TILE_Q,),
        in_specs=[
            pl.BlockSpec((1, TILE_Q, 3), lambda qt: (b, qt, 0)),
            pl.BlockSpec((1, 3, N1), lambda qt: (b, 0, 0)),
        ],
        out_specs=[
            pl.BlockSpec((TILE_Q, 3), lambda qt: (qt, 0)),
            pl.BlockSpec((TILE_Q, 48), lambda qt: (qt, 0)),
        ],
        out_shape=[
            jax.ShapeDtypeStruct((N2, 3), jnp.int32),
            jax.ShapeDtypeStruct((N2, 48), jnp.float32),
        ],
    )(p2, p1t)


def _gather_body(table_hbm, idx_hbm, wts_hbm, out_hbm,
                 idx_v0, idx_v1, wts_v0, wts_v1, rows_v0, rows_v1, out_v,
                 sem0, sem1):
    wid = lax.axis_index("s") * 2 + lax.axis_index("c")
    qbase = wid * QPW
    idx_v = [idx_v0, idx_v1]
    wts_v = [wts_v0, wts_v1]
    rows_v = [rows_v0, rows_v1]
    sems = [sem0, sem1]

    def start(ch):
        q0 = qbase + ch * CQ
        s = ch % 2
        pltpu.sync_copy(idx_hbm.at[pl.ds(q0 * 3, 3 * CQ)], idx_v[s])
        pltpu.sync_copy(wts_hbm.at[pl.ds(q0, CQ)], wts_v[s])
        return pltpu.async_copy(table_hbm.at[idx_v[s]], rows_v[s], sems[s])

    cp = start(0)
    for ch in range(NCH):
        nxt_cp = start(ch + 1) if ch + 1 < NCH else None
        cp.wait()
        s = ch % 2
        wv, rv = wts_v[s], rows_v[s]

        def qbody(q, _):
            w0 = wv[q, pl.ds(0, 16)]
            w1 = wv[q, pl.ds(16, 16)]
            w2 = wv[q, pl.ds(32, 16)]
            for c in range(C1 // 16):
                sl = pl.ds(c * 16, 16)
                out_v[q, sl] = (rv[3 * q, sl] * w0
                                + rv[3 * q + 1, sl] * w1
                                + rv[3 * q + 2, sl] * w2)
            return 0

        lax.fori_loop(0, CQ, qbody, 0)
        pltpu.sync_copy(out_v, out_hbm.at[pl.ds(qbase + ch * CQ, CQ)])
        cp = nxt_cp


@functools.lru_cache(maxsize=None)
def _make_gather_interp():
    return pl.kernel(
        _gather_body,
        out_type=jax.ShapeDtypeStruct((N2, C1), jnp.float32),
        mesh=plsc.VectorSubcoreMesh(core_axis_name="c", subcore_axis_name="s"),
        scratch_types=[
            pltpu.VMEM((3 * CQ,), jnp.int32),
            pltpu.VMEM((3 * CQ,), jnp.int32),
            pltpu.VMEM((CQ, 48), jnp.float32),
            pltpu.VMEM((CQ, 48), jnp.float32),
            pltpu.VMEM((3 * CQ, C1), jnp.float32),
            pltpu.VMEM((3 * CQ, C1), jnp.float32),
            pltpu.VMEM((CQ, C1), jnp.float32),
            pltpu.SemaphoreType.DMA,
            pltpu.SemaphoreType.DMA,
        ],
    )


def _mlp_body(out_in_ref, interp_ref, x2_ref, w1a_ref, w1b_ref, t1_ref,
              w2_ref, t2_ref, out_ref):
    del out_in_ref
    z = jnp.dot(interp_ref[...], w1a_ref[...],
                preferred_element_type=jnp.float32)
    z = z + jnp.dot(x2_ref[0], w1b_ref[...],
                    preferred_element_type=jnp.float32)
    h = jnp.maximum(z + t1_ref[...], 0.0)
    out_ref[0] = jnp.dot(h, w2_ref[...],
                         preferred_element_type=jnp.float32) + t2_ref[...]


def _mlp(out_in, interp, x2, w1a, w1b, t1, w2t, t2, b):
    return pl.pallas_call(
        _mlp_body,
        grid=(N2 // TILE_M,),
        in_specs=[
            pl.BlockSpec((1, TILE_M, FEA_OUT), lambda i: (b, i, 0)),
            pl.BlockSpec((TILE_M, C1), lambda i: (i, 0)),
            pl.BlockSpec((1, TILE_M, C2), lambda i: (b, i, 0)),
            pl.BlockSpec((C1, FEA_OUT), lambda i: (0, 0)),
            pl.BlockSpec((C2, FEA_OUT), lambda i: (0, 0)),
            pl.BlockSpec((1, FEA_OUT), lambda i: (0, 0)),
            pl.BlockSpec((FEA_OUT, FEA_OUT), lambda i: (0, 0)),
            pl.BlockSpec((1, FEA_OUT), lambda i: (0, 0)),
        ],
        out_specs=pl.BlockSpec((1, TILE_M, FEA_OUT), lambda i: (b, i, 0)),
        out_shape=jax.ShapeDtypeStruct((B, N2, FEA_OUT), jnp.float32),
        input_output_aliases={0: 0},
    )(out_in, interp, x2, w1a, w1b, t1, w2t, t2)


def kernel(x1, p1, x2, p2, W1, b1, g1, be1, m1, v1, W2, b2, g2, be2, m2, v2):
    p1t = jnp.swapaxes(p1, 1, 2)
    table = x1.reshape(B * N1, C1)

    s1 = g1 / jnp.sqrt(v1 + 1e-5)
    t1 = (b1 - m1) * s1 + be1
    s2 = g2 / jnp.sqrt(v2 + 1e-5)
    t2 = (b2 - m2) * s2 + be2
    w1s = W1 * s1[:, None]
    w1a = jnp.transpose(w1s[:, :C1])
    w1b = jnp.transpose(w1s[:, C1:])
    w2t = jnp.transpose(W2 * s2[:, None])
    t1 = t1.reshape(1, FEA_OUT)
    t2 = t2.reshape(1, FEA_OUT)

    gather = _make_gather_interp()
    out = None
    for b in range(B):
        idx_b, wts_b = _knn(p2, p1t, b)
        interp_b = gather(table, idx_b.reshape(-1), wts_b)
        if out is None:
            out = _mlp(jnp.zeros((B, N2, FEA_OUT), jnp.float32),
                       interp_b, x2, w1a, w1b, t1, w2t, t2, b)
        else:
            out = _mlp(out, interp_b, x2, w1a, w1b, t1, w2t, t2, b)
    return out, p2


# speedup vs baseline: 1.5046x; 1.0128x over previous
import functools

import jax
import jax.numpy as jnp
from jax import lax
from jax.experimental import pallas as pl
from jax.experimental.pallas import tpu as pltpu
from jax.experimental.pallas import tpu_sc as plsc

B, N1, N2 = 4, 1024, 4096
C1, C2 = 256, 256
FEA_IN, FEA_OUT = 512, 256

TILE_Q = 1024
TILE_M = 1024

NW = 32
QPW = N2 // NW
CQ = 32
NCH = QPW // CQ


def _make_knn_body(b):
    def _knn_body(p2_ref, p1t_ref, idx_ref, w_ref):
        d2 = None
        for d in range(3):
            q = p2_ref[0, :, d:d + 1]
            r = p1t_ref[0, d:d + 1, :]
            diff = q - r
            sq = diff * diff
            d2 = sq if d2 is None else d2 + sq
        iota = lax.broadcasted_iota(jnp.int32, (TILE_Q, N1), 1).astype(
            jnp.float32)
        inf = jnp.float32(jnp.inf)
        vals, idxs = [], []
        d = d2
        for _ in range(3):
            m = jnp.min(d, axis=1, keepdims=True)
            i = jnp.min(jnp.where(d == m, iota, jnp.float32(N1)),
                        axis=1, keepdims=True)
            vals.append(m)
            idxs.append(i.astype(jnp.int32))
            d = jnp.where(iota == i, inf, d)
        r0 = 1.0 / (jnp.sqrt(vals[0]) + 1e-8)
        r1 = 1.0 / (jnp.sqrt(vals[1]) + 1e-8)
        r2 = 1.0 / (jnp.sqrt(vals[2]) + 1e-8)
        norm = r0 + r1 + r2
        w_ref[...] = jnp.concatenate(
            [jnp.broadcast_to(r0 / norm, (TILE_Q, 16)),
             jnp.broadcast_to(r1 / norm, (TILE_Q, 16)),
             jnp.broadcast_to(r2 / norm, (TILE_Q, 16))], axis=1)
        base = b * N1
        idx_ref[...] = jnp.concatenate(
            [idxs[0] + base, idxs[1] + base, idxs[2] + base], axis=1)
    return _knn_body


def _knn(p2, p1t, b):
    return pl.pallas_call(
        _make_knn_body(b),
        grid=(N2 // TILE_Q,),
        in_specs=[
            pl.BlockSpec((1, TILE_Q, 3), lambda qt: (b, qt, 0)),
            pl.BlockSpec((1, 3, N1), lambda qt: (b, 0, 0)),
        ],
        out_specs=[
            pl.BlockSpec((TILE_Q, 3), lambda qt: (qt, 0)),
            pl.BlockSpec((TILE_Q, 48), lambda qt: (qt, 0)),
        ],
        out_shape=[
            jax.ShapeDtypeStruct((N2, 3), jnp.int32),
            jax.ShapeDtypeStruct((N2, 48), jnp.float32),
        ],
    )(p2, p1t)


def _gather_body(table_hbm, idx_hbm, wts_hbm, out_hbm,
                 idx_v0, idx_v1, wts_v0, wts_v1, rows_v0, rows_v1, out_v,
                 sem0, sem1):
    wid = lax.axis_index("s") * 2 + lax.axis_index("c")
    qbase = wid * QPW
    idx_v = [idx_v0, idx_v1]
    wts_v = [wts_v0, wts_v1]
    rows_v = [rows_v0, rows_v1]
    sems = [sem0, sem1]

    def start(ch):
        q0 = qbase + ch * CQ
        s = ch % 2
        pltpu.sync_copy(idx_hbm.at[pl.ds(q0 * 3, 3 * CQ)], idx_v[s])
        pltpu.sync_copy(wts_hbm.at[pl.ds(q0, CQ)], wts_v[s])
        return pltpu.async_copy(table_hbm.at[idx_v[s]], rows_v[s], sems[s])

    cp = start(0)
    for ch in range(NCH):
        nxt_cp = start(ch + 1) if ch + 1 < NCH else None
        cp.wait()
        s = ch % 2
        wv, rv = wts_v[s], rows_v[s]

        def qbody(q, _):
            w0 = wv[q, pl.ds(0, 16)]
            w1 = wv[q, pl.ds(16, 16)]
            w2 = wv[q, pl.ds(32, 16)]
            for c in range(C1 // 16):
                sl = pl.ds(c * 16, 16)
                out_v[q, sl] = (rv[3 * q, sl] * w0
                                + rv[3 * q + 1, sl] * w1
                                + rv[3 * q + 2, sl] * w2)
            return 0

        lax.fori_loop(0, CQ, qbody, 0)
        pltpu.sync_copy(out_v, out_hbm.at[pl.ds(qbase + ch * CQ, CQ)])
        cp = nxt_cp


@functools.lru_cache(maxsize=None)
def _make_gather_interp():
    return pl.kernel(
        _gather_body,
        out_type=jax.ShapeDtypeStruct((N2, C1), jnp.float32),
        mesh=plsc.VectorSubcoreMesh(core_axis_name="c", subcore_axis_name="s"),
        scratch_types=[
            pltpu.VMEM((3 * CQ,), jnp.int32),
            pltpu.VMEM((3 * CQ,), jnp.int32),
            pltpu.VMEM((CQ, 48), jnp.float32),
            pltpu.VMEM((CQ, 48), jnp.float32),
            pltpu.VMEM((3 * CQ, C1), jnp.float32),
            pltpu.VMEM((3 * CQ, C1), jnp.float32),
            pltpu.VMEM((CQ, C1), jnp.float32),
            pltpu.SemaphoreType.DMA,
            pltpu.SemaphoreType.DMA,
        ],
    )


def _mlp_compute(interp_ref, x2_ref, w1a_ref, w1b_ref, t1_ref, w2_ref,
                 t2_ref, out_ref):
    z = jnp.dot(interp_ref[...], w1a_ref[...],
                preferred_element_type=jnp.float32)
    z = z + jnp.dot(x2_ref[0], w1b_ref[...],
                    preferred_element_type=jnp.float32)
    h = jnp.maximum(z + t1_ref[...], 0.0)
    out_ref[0] = jnp.dot(h, w2_ref[...],
                         preferred_element_type=jnp.float32) + t2_ref[...]


def _mlp_body_first(interp_ref, x2_ref, w1a_ref, w1b_ref, t1_ref,
                    w2_ref, t2_ref, out_ref):
    _mlp_compute(interp_ref, x2_ref, w1a_ref, w1b_ref, t1_ref, w2_ref,
                 t2_ref, out_ref)


def _mlp_body_chain(out_in_ref, interp_ref, x2_ref, w1a_ref, w1b_ref, t1_ref,
                    w2_ref, t2_ref, out_ref):
    del out_in_ref
    _mlp_compute(interp_ref, x2_ref, w1a_ref, w1b_ref, t1_ref, w2_ref,
                 t2_ref, out_ref)


def _mlp(out_in, interp, x2, w1a, w1b, t1, w2t, t2, b):
    specs = [
        pl.BlockSpec((TILE_M, C1), lambda i: (i, 0)),
        pl.BlockSpec((1, TILE_M, C2), lambda i: (b, i, 0)),
        pl.BlockSpec((C1, FEA_OUT), lambda i: (0, 0)),
        pl.BlockSpec((C2, FEA_OUT), lambda i: (0, 0)),
        pl.BlockSpec((1, FEA_OUT), lambda i: (0, 0)),
        pl.BlockSpec((FEA_OUT, FEA_OUT), lambda i: (0, 0)),
        pl.BlockSpec((1, FEA_OUT), lambda i: (0, 0)),
    ]
    out_spec = pl.BlockSpec((1, TILE_M, FEA_OUT), lambda i: (b, i, 0))
    out_shape = jax.ShapeDtypeStruct((B, N2, FEA_OUT), jnp.float32)
    if out_in is None:
        return pl.pallas_call(
            _mlp_body_first,
            grid=(N2 // TILE_M,),
            in_specs=specs,
            out_specs=out_spec,
            out_shape=out_shape,
        )(interp, x2, w1a, w1b, t1, w2t, t2)
    return pl.pallas_call(
        _mlp_body_chain,
        grid=(N2 // TILE_M,),
        in_specs=[out_spec] + specs,
        out_specs=out_spec,
        out_shape=out_shape,
        input_output_aliases={0: 0},
    )(out_in, interp, x2, w1a, w1b, t1, w2t, t2)


def kernel(x1, p1, x2, p2, W1, b1, g1, be1, m1, v1, W2, b2, g2, be2, m2, v2):
    p1t = jnp.swapaxes(p1, 1, 2)
    table = x1.reshape(B * N1, C1)

    s1 = g1 / jnp.sqrt(v1 + 1e-5)
    t1 = (b1 - m1) * s1 + be1
    s2 = g2 / jnp.sqrt(v2 + 1e-5)
    t2 = (b2 - m2) * s2 + be2
    w1s = W1 * s1[:, None]
    w1a = jnp.transpose(w1s[:, :C1])
    w1b = jnp.transpose(w1s[:, C1:])
    w2t = jnp.transpose(W2 * s2[:, None])
    t1 = t1.reshape(1, FEA_OUT)
    t2 = t2.reshape(1, FEA_OUT)

    gather = _make_gather_interp()
    out = None
    for b in range(B):
        idx_b, wts_b = _knn(p2, p1t, b)
        interp_b = gather(table, idx_b.reshape(-1), wts_b)
        out = _mlp(out, interp_b, x2, w1a, w1b, t1, w2t, t2, b)
    return out, p2


# speedup vs baseline: 1.6511x; 1.0974x over previous
import functools

import jax
import jax.numpy as jnp
from jax import lax
from jax.experimental import pallas as pl
from jax.experimental.pallas import tpu as pltpu
from jax.experimental.pallas import tpu_sc as plsc

B, N1, N2 = 4, 1024, 4096
C1, C2 = 256, 256
FEA_IN, FEA_OUT = 512, 256

TILE_Q = 1024
TILE_M = 1024

NW = 32
QPW = N2 // NW
CQ = 32
NCH = QPW // CQ


def _make_knn_body(b):
    def _knn_body(p2_ref, p1t_ref, idx_ref, w_ref):
        d2 = None
        for d in range(3):
            q = p2_ref[0, :, d:d + 1]
            r = p1t_ref[0, d:d + 1, :]
            diff = q - r
            sq = diff * diff
            d2 = sq if d2 is None else d2 + sq
        iota = lax.broadcasted_iota(jnp.int32, (TILE_Q, N1), 1).astype(
            jnp.float32)
        inf = jnp.float32(jnp.inf)
        vals, idxs = [], []
        d = d2
        for _ in range(3):
            m = jnp.min(d, axis=1, keepdims=True)
            i = jnp.min(jnp.where(d == m, iota, jnp.float32(N1)),
                        axis=1, keepdims=True)
            vals.append(m)
            idxs.append(i.astype(jnp.int32))
            d = jnp.where(iota == i, inf, d)
        r0 = 1.0 / (jnp.sqrt(vals[0]) + 1e-8)
        r1 = 1.0 / (jnp.sqrt(vals[1]) + 1e-8)
        r2 = 1.0 / (jnp.sqrt(vals[2]) + 1e-8)
        norm = r0 + r1 + r2
        w_ref[...] = jnp.concatenate(
            [jnp.broadcast_to(r0 / norm, (TILE_Q, 16)),
             jnp.broadcast_to(r1 / norm, (TILE_Q, 16)),
             jnp.broadcast_to(r2 / norm, (TILE_Q, 16))], axis=1)
        base = b * N1
        idx_ref[...] = jnp.concatenate(
            [idxs[0] + base, idxs[1] + base, idxs[2] + base], axis=1)
    return _knn_body


def _knn(p2, p1t, b):
    return pl.pallas_call(
        _make_knn_body(b),
        grid=(N2 // TILE_Q,),
        in_specs=[
            pl.BlockSpec((1, TILE_Q, 3), lambda qt: (b, qt, 0)),
            pl.BlockSpec((1, 3, N1), lambda qt: (b, 0, 0)),
        ],
        out_specs=[
            pl.BlockSpec((TILE_Q, 3), lambda qt: (qt, 0)),
            pl.BlockSpec((TILE_Q, 48), lambda qt: (qt, 0)),
        ],
        out_shape=[
            jax.ShapeDtypeStruct((N2, 3), jnp.int32),
            jax.ShapeDtypeStruct((N2, 48), jnp.float32),
        ],
    )(p2, p1t)


def _gather_body(table_hbm, idx_hbm, wts_hbm, out_hbm,
                 idx_v0, idx_v1, wts_v0, wts_v1, rows_v0, rows_v1, out_v,
                 sem0, sem1):
    wid = lax.axis_index("s") * 2 + lax.axis_index("c")
    qbase = wid * QPW
    idx_v = [idx_v0, idx_v1]
    wts_v = [wts_v0, wts_v1]
    rows_v = [rows_v0, rows_v1]
    sems = [sem0, sem1]

    def start(ch):
        q0 = qbase + ch * CQ
        s = ch % 2
        pltpu.sync_copy(idx_hbm.at[pl.ds(q0 * 3, 3 * CQ)], idx_v[s])
        pltpu.sync_copy(wts_hbm.at[pl.ds(q0, CQ)], wts_v[s])
        return pltpu.async_copy(table_hbm.at[idx_v[s]], rows_v[s], sems[s])

    cp = start(0)
    for ch in range(NCH):
        nxt_cp = start(ch + 1) if ch + 1 < NCH else None
        cp.wait()
        s = ch % 2
        wv, rv = wts_v[s], rows_v[s]

        @plsc.parallel_loop(0, CQ, unroll=4)
        def qbody(q):
            w0 = wv[q, pl.ds(0, 16)]
            w1 = wv[q, pl.ds(16, 16)]
            w2 = wv[q, pl.ds(32, 16)]
            for c in range(C1 // 16):
                sl = pl.ds(c * 16, 16)
                out_v[q, sl] = (rv[3 * q, sl] * w0
                                + rv[3 * q + 1, sl] * w1
                                + rv[3 * q + 2, sl] * w2)
        pltpu.sync_copy(out_v, out_hbm.at[pl.ds(qbase + ch * CQ, CQ)])
        cp = nxt_cp


@functools.lru_cache(maxsize=None)
def _make_gather_interp():
    return pl.kernel(
        _gather_body,
        out_type=jax.ShapeDtypeStruct((N2, C1), jnp.float32),
        mesh=plsc.VectorSubcoreMesh(core_axis_name="c", subcore_axis_name="s"),
        scratch_types=[
            pltpu.VMEM((3 * CQ,), jnp.int32),
            pltpu.VMEM((3 * CQ,), jnp.int32),
            pltpu.VMEM((CQ, 48), jnp.float32),
            pltpu.VMEM((CQ, 48), jnp.float32),
            pltpu.VMEM((3 * CQ, C1), jnp.float32),
            pltpu.VMEM((3 * CQ, C1), jnp.float32),
            pltpu.VMEM((CQ, C1), jnp.float32),
            pltpu.SemaphoreType.DMA,
            pltpu.SemaphoreType.DMA,
        ],
    )


def _mlp_compute(interp_ref, x2_ref, w1a_ref, w1b_ref, t1_ref, w2_ref,
                 t2_ref, out_ref):
    z = jnp.dot(interp_ref[...], w1a_ref[...],
                preferred_element_type=jnp.float32)
    z = z + jnp.dot(x2_ref[0], w1b_ref[...],
                    preferred_element_type=jnp.float32)
    h = jnp.maximum(z + t1_ref[...], 0.0)
    out_ref[0] = jnp.dot(h, w2_ref[...],
                         preferred_element_type=jnp.float32) + t2_ref[...]


def _mlp_body_first(interp_ref, x2_ref, w1a_ref, w1b_ref, t1_ref,
                    w2_ref, t2_ref, out_ref):
    _mlp_compute(interp_ref, x2_ref, w1a_ref, w1b_ref, t1_ref, w2_ref,
                 t2_ref, out_ref)


def _mlp_body_chain(out_in_ref, interp_ref, x2_ref, w1a_ref, w1b_ref, t1_ref,
                    w2_ref, t2_ref, out_ref):
    del out_in_ref
    _mlp_compute(interp_ref, x2_ref, w1a_ref, w1b_ref, t1_ref, w2_ref,
                 t2_ref, out_ref)


def _mlp(out_in, interp, x2, w1a, w1b, t1, w2t, t2, b):
    specs = [
        pl.BlockSpec((TILE_M, C1), lambda i: (i, 0)),
        pl.BlockSpec((1, TILE_M, C2), lambda i: (b, i, 0)),
        pl.BlockSpec((C1, FEA_OUT), lambda i: (0, 0)),
        pl.BlockSpec((C2, FEA_OUT), lambda i: (0, 0)),
        pl.BlockSpec((1, FEA_OUT), lambda i: (0, 0)),
        pl.BlockSpec((FEA_OUT, FEA_OUT), lambda i: (0, 0)),
        pl.BlockSpec((1, FEA_OUT), lambda i: (0, 0)),
    ]
    out_spec = pl.BlockSpec((1, TILE_M, FEA_OUT), lambda i: (b, i, 0))
    out_shape = jax.ShapeDtypeStruct((B, N2, FEA_OUT), jnp.float32)
    if out_in is None:
        return pl.pallas_call(
            _mlp_body_first,
            grid=(N2 // TILE_M,),
            in_specs=specs,
            out_specs=out_spec,
            out_shape=out_shape,
        )(interp, x2, w1a, w1b, t1, w2t, t2)
    return pl.pallas_call(
        _mlp_body_chain,
        grid=(N2 // TILE_M,),
        in_specs=[out_spec] + specs,
        out_specs=out_spec,
        out_shape=out_shape,
        input_output_aliases={0: 0},
    )(out_in, interp, x2, w1a, w1b, t1, w2t, t2)


def kernel(x1, p1, x2, p2, W1, b1, g1, be1, m1, v1, W2, b2, g2, be2, m2, v2):
    p1t = jnp.swapaxes(p1, 1, 2)
    table = x1.reshape(B * N1, C1)

    s1 = g1 / jnp.sqrt(v1 + 1e-5)
    t1 = (b1 - m1) * s1 + be1
    s2 = g2 / jnp.sqrt(v2 + 1e-5)
    t2 = (b2 - m2) * s2 + be2
    w1s = W1 * s1[:, None]
    w1a = jnp.transpose(w1s[:, :C1])
    w1b = jnp.transpose(w1s[:, C1:])
    w2t = jnp.transpose(W2 * s2[:, None])
    t1 = t1.reshape(1, FEA_OUT)
    t2 = t2.reshape(1, FEA_OUT)

    gather = _make_gather_interp()
    out = None
    for b in range(B):
        idx_b, wts_b = _knn(p2, p1t, b)
        interp_b = gather(table, idx_b.reshape(-1), wts_b)
        out = _mlp(out, interp_b, x2, w1a, w1b, t1, w2t, t2, b)
    return out, p2


# speedup vs baseline: 1.6934x; 1.0256x over previous
import functools

import jax
import jax.numpy as jnp
from jax import lax
from jax.experimental import pallas as pl
from jax.experimental.pallas import tpu as pltpu
from jax.experimental.pallas import tpu_sc as plsc

B, N1, N2 = 4, 1024, 4096
C1, C2 = 256, 256
FEA_IN, FEA_OUT = 512, 256

TILE_Q = 1024
TILE_M = 1024

NW = 32
QPW = N2 // NW
CQ = 32
NCH = QPW // CQ


def _make_knn_body(b):
    def _knn_body(p2_ref, p1t_ref, idx_ref, w_ref):
        d2 = None
        for d in range(3):
            q = p2_ref[0, :, d:d + 1]
            r = p1t_ref[0, d:d + 1, :]
            diff = q - r
            sq = diff * diff
            d2 = sq if d2 is None else d2 + sq
        iota = lax.broadcasted_iota(jnp.int32, (TILE_Q, N1), 1).astype(
            jnp.float32)
        inf = jnp.float32(jnp.inf)
        vals, idxs = [], []
        d = d2
        for _ in range(3):
            m = jnp.min(d, axis=1, keepdims=True)
            i = jnp.min(jnp.where(d == m, iota, jnp.float32(N1)),
                        axis=1, keepdims=True)
            vals.append(m)
            idxs.append(i.astype(jnp.int32))
            d = jnp.where(iota == i, inf, d)
        r0 = 1.0 / (jnp.sqrt(vals[0]) + 1e-8)
        r1 = 1.0 / (jnp.sqrt(vals[1]) + 1e-8)
        r2 = 1.0 / (jnp.sqrt(vals[2]) + 1e-8)
        norm = r0 + r1 + r2
        w_ref[...] = jnp.concatenate(
            [jnp.broadcast_to(r0 / norm, (TILE_Q, 16)),
             jnp.broadcast_to(r1 / norm, (TILE_Q, 16)),
             jnp.broadcast_to(r2 / norm, (TILE_Q, 16))], axis=1)
        base = b * N1
        idx_ref[...] = jnp.concatenate(
            [idxs[0] + base, idxs[1] + base, idxs[2] + base], axis=1)
    return _knn_body


def _knn(p2, p1t, b):
    return pl.pallas_call(
        _make_knn_body(b),
        grid=(N2 // TILE_Q,),
        in_specs=[
            pl.BlockSpec((1, TILE_Q, 3), lambda qt: (b, qt, 0)),
            pl.BlockSpec((1, 3, N1), lambda qt: (b, 0, 0)),
        ],
        out_specs=[
            pl.BlockSpec((TILE_Q, 3), lambda qt: (qt, 0)),
            pl.BlockSpec((TILE_Q, 48), lambda qt: (qt, 0)),
        ],
        out_shape=[
            jax.ShapeDtypeStruct((N2, 3), jnp.int32),
            jax.ShapeDtypeStruct((N2, 48), jnp.float32),
        ],
    )(p2, p1t)


def _gather_body(table_hbm, idx_hbm, wts_hbm, out_hbm,
                 idx_v0, idx_v1, wts_v0, wts_v1, rows_v0, rows_v1, out_v,
                 sem0, sem1):
    wid = lax.axis_index("s") * 2 + lax.axis_index("c")
    qbase = wid * QPW
    idx_v = [idx_v0, idx_v1]
    wts_v = [wts_v0, wts_v1]
    rows_v = [rows_v0, rows_v1]
    sems = [sem0, sem1]

    def start(ch):
        q0 = qbase + ch * CQ
        s = ch % 2
        pltpu.sync_copy(idx_hbm.at[pl.ds(q0 * 3, 3 * CQ)], idx_v[s])
        pltpu.sync_copy(wts_hbm.at[pl.ds(q0, CQ)], wts_v[s])
        return pltpu.async_copy(table_hbm.at[idx_v[s]], rows_v[s], sems[s])

    cp = start(0)
    for ch in range(NCH):
        nxt_cp = start(ch + 1) if ch + 1 < NCH else None
        cp.wait()
        s = ch % 2
        wv, rv = wts_v[s], rows_v[s]

        @plsc.parallel_loop(0, CQ, unroll=8)
        def qbody(q):
            w0 = wv[q, pl.ds(0, 16)]
            w1 = wv[q, pl.ds(16, 16)]
            w2 = wv[q, pl.ds(32, 16)]
            for c in range(C1 // 16):
                sl = pl.ds(c * 16, 16)
                out_v[q, sl] = (rv[3 * q, sl] * w0
                                + rv[3 * q + 1, sl] * w1
                                + rv[3 * q + 2, sl] * w2)
        pltpu.sync_copy(out_v, out_hbm.at[pl.ds(qbase + ch * CQ, CQ)])
        cp = nxt_cp


@functools.lru_cache(maxsize=None)
def _make_gather_interp():
    return pl.kernel(
        _gather_body,
        out_type=jax.ShapeDtypeStruct((N2, C1), jnp.float32),
        mesh=plsc.VectorSubcoreMesh(core_axis_name="c", subcore_axis_name="s"),
        scratch_types=[
            pltpu.VMEM((3 * CQ,), jnp.int32),
            pltpu.VMEM((3 * CQ,), jnp.int32),
            pltpu.VMEM((CQ, 48), jnp.float32),
            pltpu.VMEM((CQ, 48), jnp.float32),
            pltpu.VMEM((3 * CQ, C1), jnp.float32),
            pltpu.VMEM((3 * CQ, C1), jnp.float32),
            pltpu.VMEM((CQ, C1), jnp.float32),
            pltpu.SemaphoreType.DMA,
            pltpu.SemaphoreType.DMA,
        ],
    )


def _mlp_compute(interp_ref, x2_ref, w1a_ref, w1b_ref, t1_ref, w2_ref,
                 t2_ref, out_ref):
    z = jnp.dot(interp_ref[...], w1a_ref[...],
                preferred_element_type=jnp.float32)
    z = z + jnp.dot(x2_ref[0], w1b_ref[...],
                    preferred_element_type=jnp.float32)
    h = jnp.maximum(z + t1_ref[...], 0.0)
    out_ref[0] = jnp.dot(h, w2_ref[...],
                         preferred_element_type=jnp.float32) + t2_ref[...]


def _mlp_body_first(interp_ref, x2_ref, w1a_ref, w1b_ref, t1_ref,
                    w2_ref, t2_ref, out_ref):
    _mlp_compute(interp_ref, x2_ref, w1a_ref, w1b_ref, t1_ref, w2_ref,
                 t2_ref, out_ref)


def _mlp_body_chain(out_in_ref, interp_ref, x2_ref, w1a_ref, w1b_ref, t1_ref,
                    w2_ref, t2_ref, out_ref):
    del out_in_ref
    _mlp_compute(interp_ref, x2_ref, w1a_ref, w1b_ref, t1_ref, w2_ref,
                 t2_ref, out_ref)


def _mlp(out_in, interp, x2, w1a, w1b, t1, w2t, t2, b):
    specs = [
        pl.BlockSpec((TILE_M, C1), lambda i: (i, 0)),
        pl.BlockSpec((1, TILE_M, C2), lambda i: (b, i, 0)),
        pl.BlockSpec((C1, FEA_OUT), lambda i: (0, 0)),
        pl.BlockSpec((C2, FEA_OUT), lambda i: (0, 0)),
        pl.BlockSpec((1, FEA_OUT), lambda i: (0, 0)),
        pl.BlockSpec((FEA_OUT, FEA_OUT), lambda i: (0, 0)),
        pl.BlockSpec((1, FEA_OUT), lambda i: (0, 0)),
    ]
    out_spec = pl.BlockSpec((1, TILE_M, FEA_OUT), lambda i: (b, i, 0))
    out_shape = jax.ShapeDtypeStruct((B, N2, FEA_OUT), jnp.float32)
    if out_in is None:
        return pl.pallas_call(
            _mlp_body_first,
            grid=(N2 // TILE_M,),
            in_specs=specs,
            out_specs=out_spec,
            out_shape=out_shape,
        )(interp, x2, w1a, w1b, t1, w2t, t2)
    return pl.pallas_call(
        _mlp_body_chain,
        grid=(N2 // TILE_M,),
        in_specs=[out_spec] + specs,
        out_specs=out_spec,
        out_shape=out_shape,
        input_output_aliases={0: 0},
    )(out_in, interp, x2, w1a, w1b, t1, w2t, t2)


def kernel(x1, p1, x2, p2, W1, b1, g1, be1, m1, v1, W2, b2, g2, be2, m2, v2):
    p1t = jnp.swapaxes(p1, 1, 2)
    table = x1.reshape(B * N1, C1)

    s1 = g1 / jnp.sqrt(v1 + 1e-5)
    t1 = (b1 - m1) * s1 + be1
    s2 = g2 / jnp.sqrt(v2 + 1e-5)
    t2 = (b2 - m2) * s2 + be2
    w1s = W1 * s1[:, None]
    w1a = jnp.transpose(w1s[:, :C1])
    w1b = jnp.transpose(w1s[:, C1:])
    w2t = jnp.transpose(W2 * s2[:, None])
    t1 = t1.reshape(1, FEA_OUT)
    t2 = t2.reshape(1, FEA_OUT)

    gather = _make_gather_interp()
    out = None
    for b in range(B):
        idx_b, wts_b = _knn(p2, p1t, b)
        interp_b = gather(table, idx_b.reshape(-1), wts_b)
        out = _mlp(out, interp_b, x2, w1a, w1b, t1, w2t, t2, b)
    return out, p2
